# baseline (speedup 1.0000x reference)
import functools

import jax
import jax.numpy as jnp
from jax import lax
from jax.experimental import pallas as pl
from jax.experimental.pallas import tpu as pltpu
from jax.experimental.pallas import tpu_sc as plsc

N = 10000
E = 160000
HID = 256

NTILES = 32
RANGE = 320
NPAD = RANGE * NTILES
CHUNK = 64


def _agg_body(table_hbm, asrt_hbm, ssrt_hbm, ld_hbm, bounds_hbm, out_hbm,
              bounds_v, sidx_v, a_v, ld_v, rows_v, acc_v, sem, *, head, twid):
    wid = lax.axis_index("s") * 2 + lax.axis_index("c")
    node_base = wid * RANGE

    zero = jnp.zeros((16,), jnp.float32)

    def _zero_row(r, _):
        def _zero_col(v, _):
            acc_v[r, pl.ds(v * 16, 16)] = zero
            return 0
        return lax.fori_loop(0, 16, _zero_col, 0)

    lax.fori_loop(0, RANGE, _zero_row, 0)

    pltpu.sync_copy(bounds_hbm, bounds_v.at[pl.ds(0, 40)])
    lo = bounds_v[pl.ds(wid, 16)][0]
    hi = bounds_v[pl.ds(wid + 1, 16)][0]
    lo8 = (lo // 8) * 8
    nchunks = (hi - lo8 + CHUNK - 1) // CHUNK

    def _chunk(k, _):
        c0 = lo8 + k * CHUNK
        pltpu.sync_copy(ssrt_hbm.at[pl.ds(c0, CHUNK)], sidx_v)
        pltpu.sync_copy(asrt_hbm.at[pl.ds(c0, CHUNK)], a_v.at[pl.ds(0, CHUNK)])
        pltpu.sync_copy(ld_hbm.at[pl.ds(c0, CHUNK)], ld_v.at[pl.ds(0, CHUNK)])
        pltpu.async_copy(table_hbm.at[sidx_v], rows_v, sem).wait()
        start_i = jnp.maximum(lo, c0)
        end_i = jnp.minimum(hi, c0 + CHUNK)

        def _edge(i, _):
            j = i - c0
            a = a_v[pl.ds(j, 16)][0]
            ldi = ld_v[pl.ds(j, 16)][0]
            for v in range(16):
                plsc.addupdate(
                    acc_v.at[ldi, pl.ds(v * 16, 16)],
                    a * rows_v[j, pl.ds(head * HID + v * 16, 16)],
                )
            return 0

        lax.fori_loop(start_i, end_i, _edge, 0)
        return 0

    lax.fori_loop(0, nchunks, _chunk, 0)
    pltpu.sync_copy(acc_v, out_hbm.at[pl.ds(node_base, RANGE)])


@functools.partial(jax.jit, static_argnames=("head", "twid"))
def _sc_aggregate(table, asrt, ssrt, ld, bounds, head=0, twid=HID):
    mesh = plsc.VectorSubcoreMesh(core_axis_name="c", subcore_axis_name="s")
    k = pl.kernel(
        functools.partial(_agg_body, head=head, twid=twid),
        mesh=mesh,
        out_type=jax.ShapeDtypeStruct((NPAD, HID), jnp.float32),
        scratch_types=[
            pltpu.VMEM((56,), jnp.int32),
            pltpu.VMEM((CHUNK,), jnp.int32),
            pltpu.VMEM((CHUNK + 16,), jnp.float32),
            pltpu.VMEM((CHUNK + 16,), jnp.int32),
            pltpu.VMEM((CHUNK, twid), jnp.float32),
            pltpu.VMEM((RANGE, HID), jnp.float32),
            pltpu.SemaphoreType.DMA,
        ],
    )
    return k(table, asrt, ssrt, ld, bounds)


def _pad_chunk(a):
    return jnp.concatenate([a, jnp.zeros((CHUNK,), a.dtype)])


def _edge_plan(dst):
    perm = jnp.argsort(dst, stable=True)
    dsts = dst[perm]
    starts = jnp.arange(NTILES + 1, dtype=jnp.int32) * RANGE
    bounds = jnp.searchsorted(dsts, starts, side="left").astype(jnp.int32)
    bounds = jnp.concatenate([bounds, jnp.zeros((7,), jnp.int32)])
    ld = _pad_chunk((dsts % RANGE).astype(jnp.int32))
    return perm, bounds, ld


def kernel(x, edge_index, W1, att_src1, att_dst1, b1, g1, be1, W2, att_src2, att_dst2, b2, g2, be2):
    src = edge_index[0]
    dst = edge_index[1]
    perm, bounds, ld = _edge_plan(dst)
    ssrt = _pad_chunk(src[perm].astype(jnp.int32))

    hW = x @ W1
    h = hW.reshape(N, 2, HID)
    alpha_s = jnp.sum(h * att_src1[None, :, :], axis=-1)
    alpha_d = jnp.sum(h * att_dst1[None, :, :], axis=-1)
    e = jax.nn.leaky_relu(alpha_s[src] + alpha_d[dst], negative_slope=0.2)
    m = jax.ops.segment_max(e, dst, num_segments=N)
    ex = jnp.exp(e - m[dst])
    s = jax.ops.segment_sum(ex, dst, num_segments=N)
    alpha = ex / (s[dst] + 1e-16)
    a_sorted = alpha[perm]
    agg0 = _sc_aggregate(hW, _pad_chunk(a_sorted[:, 0]), ssrt, ld, bounds,
                         head=0, twid=2 * HID)[:N]
    agg1 = _sc_aggregate(hW, _pad_chunk(a_sorted[:, 1]), ssrt, ld, bounds,
                         head=1, twid=2 * HID)[:N]
    hcat = jnp.concatenate([agg0, agg1], axis=1) + b1

    mu1 = hcat.mean(axis=0, keepdims=True)
    var1 = hcat.var(axis=0, keepdims=True)
    hbn = (hcat - mu1) / jnp.sqrt(var1 + 1e-5) * g1 + be1
    hr = jax.nn.relu(hbn)

    h2 = (hr @ W2).reshape(N, 1, HID)
    alpha_s2 = jnp.sum(h2 * att_src2[None, :, :], axis=-1)
    alpha_d2 = jnp.sum(h2 * att_dst2[None, :, :], axis=-1)
    e2 = jax.nn.leaky_relu(alpha_s2[src] + alpha_d2[dst], negative_slope=0.2)
    m2 = jax.ops.segment_max(e2, dst, num_segments=N)
    ex2 = jnp.exp(e2 - m2[dst])
    s2 = jax.ops.segment_sum(ex2, dst, num_segments=N)
    alpha2 = ex2 / (s2[dst] + 1e-16)
    agg2 = jax.ops.segment_sum(alpha2[:, :, None] * h2[src], dst, num_segments=N)
    ho2 = agg2.mean(axis=1) + b2

    mu2 = ho2.mean(axis=0, keepdims=True)
    var2 = ho2.var(axis=0, keepdims=True)
    y = (ho2 - mu2) / jnp.sqrt(var2 + 1e-5) * g2 + be2
    out = y.mean(axis=0, keepdims=True)
    return out


# speedup vs baseline: 1.0268x; 1.0268x over previous
import functools

import jax
import jax.numpy as jnp
from jax import lax
from jax.experimental import pallas as pl
from jax.experimental.pallas import tpu as pltpu
from jax.experimental.pallas import tpu_sc as plsc

N = 10000
E = 160000
HID = 256

NTILES = 32
RANGE = 320
NPAD = RANGE * NTILES
CHUNK = 128


def _agg_body(table_hbm, asrt_hbm, ssrt_hbm, ld_hbm, bounds_hbm, out_hbm,
              bounds_v, sidx_v, a_v, ld_v, rows_v, acc_v, sem):
    wid = lax.axis_index("s") * 2 + lax.axis_index("c")
    node_base = wid * RANGE

    zero = jnp.zeros((16,), jnp.float32)

    def _zero_row(r, _):
        def _zero_col(v, _):
            acc_v[r, pl.ds(v * 16, 16)] = zero
            return 0
        return lax.fori_loop(0, 16, _zero_col, 0)

    lax.fori_loop(0, RANGE, _zero_row, 0)

    pltpu.sync_copy(bounds_hbm, bounds_v.at[pl.ds(0, 40)])
    lo = bounds_v[pl.ds(wid, 16)][0]
    hi = bounds_v[pl.ds(wid + 1, 16)][0]
    lo8 = (lo // 8) * 8
    nchunks = (hi - lo8 + CHUNK - 1) // CHUNK

    def _chunk(k, _):
        c0 = lo8 + k * CHUNK
        pltpu.sync_copy(ssrt_hbm.at[pl.ds(c0, CHUNK)], sidx_v)
        pltpu.sync_copy(asrt_hbm.at[pl.ds(c0, CHUNK)], a_v.at[pl.ds(0, CHUNK)])
        pltpu.sync_copy(ld_hbm.at[pl.ds(c0, CHUNK)], ld_v.at[pl.ds(0, CHUNK)])
        pltpu.async_copy(table_hbm.at[sidx_v], rows_v, sem).wait()
        start_i = jnp.maximum(lo, c0)
        end_i = jnp.minimum(hi, c0 + CHUNK)

        def _edge(i, _):
            j = i - c0
            a = a_v[pl.ds(j, 16)][0]
            ldi = ld_v[pl.ds(j, 16)][0]
            for v in range(16):
                sl = pl.ds(v * 16, 16)
                plsc.addupdate(acc_v.at[ldi, sl], a * rows_v[j, sl])
            return 0

        lax.fori_loop(start_i, end_i, _edge, 0)
        return 0

    lax.fori_loop(0, nchunks, _chunk, 0)
    pltpu.sync_copy(acc_v, out_hbm.at[pl.ds(node_base, RANGE)])


@jax.jit
def _sc_aggregate(table, asrt, ssrt, ld, bounds):
    mesh = plsc.VectorSubcoreMesh(core_axis_name="c", subcore_axis_name="s")
    k = pl.kernel(
        _agg_body,
        mesh=mesh,
        out_type=jax.ShapeDtypeStruct((NPAD, HID), jnp.float32),
        scratch_types=[
            pltpu.VMEM((56,), jnp.int32),
            pltpu.VMEM((CHUNK,), jnp.int32),
            pltpu.VMEM((CHUNK + 16,), jnp.float32),
            pltpu.VMEM((CHUNK + 16,), jnp.int32),
            pltpu.VMEM((CHUNK, HID), jnp.float32),
            pltpu.VMEM((RANGE, HID), jnp.float32),
            pltpu.SemaphoreType.DMA,
        ],
    )
    return k(table, asrt, ssrt, ld, bounds)


def _pad_chunk(a):
    return jnp.concatenate([a, jnp.zeros((CHUNK,), a.dtype)])


def _edge_plan(dst):
    perm = jnp.argsort(dst, stable=True)
    dsts = dst[perm]
    starts = jnp.arange(NTILES + 1, dtype=jnp.int32) * RANGE
    bounds = jnp.searchsorted(dsts, starts, side="left").astype(jnp.int32)
    bounds = jnp.concatenate([bounds, jnp.zeros((7,), jnp.int32)])
    ld = _pad_chunk((dsts % RANGE).astype(jnp.int32))
    return perm, bounds, ld


def kernel(x, edge_index, W1, att_src1, att_dst1, b1, g1, be1, W2, att_src2, att_dst2, b2, g2, be2):
    src = edge_index[0]
    dst = edge_index[1]
    perm, bounds, ld = _edge_plan(dst)
    ssrt = _pad_chunk(src[perm].astype(jnp.int32))

    hW = x @ W1
    h = hW.reshape(N, 2, HID)
    alpha_s = jnp.sum(h * att_src1[None, :, :], axis=-1)
    alpha_d = jnp.sum(h * att_dst1[None, :, :], axis=-1)
    e = jax.nn.leaky_relu(alpha_s[src] + alpha_d[dst], negative_slope=0.2)
    m = jax.ops.segment_max(e, dst, num_segments=N)
    ex = jnp.exp(e - m[dst])
    s = jax.ops.segment_sum(ex, dst, num_segments=N)
    alpha = ex / (s[dst] + 1e-16)
    a_sorted = alpha[perm]
    hW2 = hW.reshape(2 * N, HID)
    agg0 = _sc_aggregate(hW2, _pad_chunk(a_sorted[:, 0]), 2 * ssrt, ld, bounds)[:N]
    agg1 = _sc_aggregate(hW2, _pad_chunk(a_sorted[:, 1]), 2 * ssrt + 1, ld, bounds)[:N]
    hcat = jnp.concatenate([agg0, agg1], axis=1) + b1

    mu1 = hcat.mean(axis=0, keepdims=True)
    var1 = hcat.var(axis=0, keepdims=True)
    hbn = (hcat - mu1) / jnp.sqrt(var1 + 1e-5) * g1 + be1
    hr = jax.nn.relu(hbn)

    h2 = (hr @ W2).reshape(N, 1, HID)
    alpha_s2 = jnp.sum(h2 * att_src2[None, :, :], axis=-1)
    alpha_d2 = jnp.sum(h2 * att_dst2[None, :, :], axis=-1)
    e2 = jax.nn.leaky_relu(alpha_s2[src] + alpha_d2[dst], negative_slope=0.2)
    m2 = jax.ops.segment_max(e2, dst, num_segments=N)
    ex2 = jnp.exp(e2 - m2[dst])
    s2 = jax.ops.segment_sum(ex2, dst, num_segments=N)
    alpha2 = ex2 / (s2[dst] + 1e-16)
    agg2 = jax.ops.segment_sum(alpha2[:, :, None] * h2[src], dst, num_segments=N)
    ho2 = agg2.mean(axis=1) + b2

    mu2 = ho2.mean(axis=0, keepdims=True)
    var2 = ho2.var(axis=0, keepdims=True)
    y = (ho2 - mu2) / jnp.sqrt(var2 + 1e-5) * g2 + be2
    out = y.mean(axis=0, keepdims=True)
    return out


# speedup vs baseline: 1.0872x; 1.0588x over previous
import functools

import jax
import jax.numpy as jnp
from jax import lax
from jax.experimental import pallas as pl
from jax.experimental.pallas import tpu as pltpu
from jax.experimental.pallas import tpu_sc as plsc

N = 10000
E = 160000
HID = 256

NTILES = 32
RANGE = 320
NPAD = RANGE * NTILES
CHUNK = 128


def _agg_body(table_hbm, asrt_hbm, ssrt_hbm, ld_hbm, bounds_hbm, out_hbm,
              bounds_v, sidx_v, a_v, ld_v, rows_v, acc_v, sem):
    wid = lax.axis_index("s") * 2 + lax.axis_index("c")
    node_base = wid * RANGE

    zero = jnp.zeros((16,), jnp.float32)

    def _zero_row(r, _):
        def _zero_col(v, _):
            acc_v[r, pl.ds(v * 16, 16)] = zero
            return 0
        return lax.fori_loop(0, 16, _zero_col, 0)

    lax.fori_loop(0, RANGE, _zero_row, 0)

    pltpu.sync_copy(bounds_hbm, bounds_v.at[pl.ds(0, 40)])
    lo = bounds_v[pl.ds(wid, 16)][0]
    hi = bounds_v[pl.ds(wid + 1, 16)][0]
    lo8 = (lo // 8) * 8
    nchunks = (hi - lo8 + CHUNK - 1) // CHUNK

    def _chunk(k, _):
        c0 = lo8 + k * CHUNK
        pltpu.sync_copy(ssrt_hbm.at[pl.ds(c0, CHUNK)], sidx_v)
        pltpu.sync_copy(asrt_hbm.at[pl.ds(c0, CHUNK)], a_v.at[pl.ds(0, CHUNK)])
        pltpu.sync_copy(ld_hbm.at[pl.ds(c0, CHUNK)], ld_v.at[pl.ds(0, CHUNK)])
        pltpu.async_copy(table_hbm.at[sidx_v], rows_v, sem).wait()
        start_i = jnp.maximum(lo, c0)
        end_i = jnp.minimum(hi, c0 + CHUNK)

        def _edge(i, _):
            j = i - c0
            a = a_v[pl.ds(j, 16)][0]
            ldi = ld_v[pl.ds(j, 16)][0]
            for v in range(16):
                sl = pl.ds(v * 16, 16)
                plsc.addupdate(acc_v.at[ldi, sl], a * rows_v[j, sl])
            return 0

        lax.fori_loop(start_i, end_i, _edge, 0)
        return 0

    lax.fori_loop(0, nchunks, _chunk, 0)
    pltpu.sync_copy(acc_v, out_hbm.at[pl.ds(node_base, RANGE)])


@jax.jit
def _sc_aggregate(table, asrt, ssrt, ld, bounds):
    mesh = plsc.VectorSubcoreMesh(core_axis_name="c", subcore_axis_name="s")
    k = pl.kernel(
        _agg_body,
        mesh=mesh,
        out_type=jax.ShapeDtypeStruct((NPAD, HID), jnp.float32),
        scratch_types=[
            pltpu.VMEM((56,), jnp.int32),
            pltpu.VMEM((CHUNK,), jnp.int32),
            pltpu.VMEM((CHUNK + 16,), jnp.float32),
            pltpu.VMEM((CHUNK + 16,), jnp.int32),
            pltpu.VMEM((CHUNK, HID), jnp.float32),
            pltpu.VMEM((RANGE, HID), jnp.float32),
            pltpu.SemaphoreType.DMA,
        ],
    )
    return k(table, asrt, ssrt, ld, bounds)


def _pad_chunk(a):
    return jnp.concatenate([a, jnp.zeros((CHUNK,), a.dtype)])


def _edge_plan(dst):
    perm = jnp.argsort(dst, stable=True)
    dsts = dst[perm]
    starts = jnp.arange(NTILES + 1, dtype=jnp.int32) * RANGE
    bounds = jnp.searchsorted(dsts, starts, side="left").astype(jnp.int32)
    bounds = jnp.concatenate([bounds, jnp.zeros((7,), jnp.int32)])
    ld = _pad_chunk((dsts % RANGE).astype(jnp.int32))
    return perm, bounds, ld


def kernel(x, edge_index, W1, att_src1, att_dst1, b1, g1, be1, W2, att_src2, att_dst2, b2, g2, be2):
    src = edge_index[0]
    dst = edge_index[1]
    perm, bounds, ld = _edge_plan(dst)
    ssrt = _pad_chunk(src[perm].astype(jnp.int32))

    hW = x @ W1
    h = hW.reshape(N, 2, HID)
    alpha_s = jnp.sum(h * att_src1[None, :, :], axis=-1)
    alpha_d = jnp.sum(h * att_dst1[None, :, :], axis=-1)
    e = jax.nn.leaky_relu(alpha_s[src] + alpha_d[dst], negative_slope=0.2)
    m = jax.ops.segment_max(e, dst, num_segments=N)
    ex = jnp.exp(e - m[dst])
    s = jax.ops.segment_sum(ex, dst, num_segments=N)
    alpha = ex / (s[dst] + 1e-16)
    a_sorted = alpha[perm]
    hW2 = hW.reshape(2 * N, HID)
    agg0 = _sc_aggregate(hW2, _pad_chunk(a_sorted[:, 0]), 2 * ssrt, ld, bounds)[:N]
    agg1 = _sc_aggregate(hW2, _pad_chunk(a_sorted[:, 1]), 2 * ssrt + 1, ld, bounds)[:N]
    hcat = jnp.concatenate([agg0, agg1], axis=1) + b1

    mu1 = hcat.mean(axis=0, keepdims=True)
    var1 = hcat.var(axis=0, keepdims=True)
    hbn = (hcat - mu1) / jnp.sqrt(var1 + 1e-5) * g1 + be1
    hr = jax.nn.relu(hbn)

    hV = hr @ W2
    h2 = hV.reshape(N, 1, HID)
    alpha_s2 = jnp.sum(h2 * att_src2[None, :, :], axis=-1)
    alpha_d2 = jnp.sum(h2 * att_dst2[None, :, :], axis=-1)
    e2 = jax.nn.leaky_relu(alpha_s2[src] + alpha_d2[dst], negative_slope=0.2)
    m2 = jax.ops.segment_max(e2, dst, num_segments=N)
    ex2 = jnp.exp(e2 - m2[dst])
    s2 = jax.ops.segment_sum(ex2, dst, num_segments=N)
    alpha2 = ex2 / (s2[dst] + 1e-16)
    a2_sorted = alpha2[perm][:, 0]
    agg2 = _sc_aggregate(hV, _pad_chunk(a2_sorted), ssrt, ld, bounds)[:N]
    ho2 = agg2[:, None, :].mean(axis=1) + b2

    mu2 = ho2.mean(axis=0, keepdims=True)
    var2 = ho2.var(axis=0, keepdims=True)
    y = (ho2 - mu2) / jnp.sqrt(var2 + 1e-5) * g2 + be2
    out = y.mean(axis=0, keepdims=True)
    return out
